# baseline (speedup 1.0000x reference)
import functools

import jax
import jax.numpy as jnp
from jax import lax
from jax.experimental import pallas as pl
from jax.experimental.pallas import tpu as pltpu
from jax.experimental.pallas import tpu_sc as plsc

NC = 2
NS = 16
LANES = 16
C = 56
SK = 8
UNR = 2 * SK


def _edge_linear(edge_attr, W, b, EP):
    E, ED = edge_attr.shape
    D = W.shape[1]
    B = 2000

    def body(a_ref, w_ref, b_ref, o_ref):
        o_ref[...] = (
            jnp.dot(a_ref[...], w_ref[...], preferred_element_type=jnp.float32)
            + b_ref[...]
        )

    return pl.pallas_call(
        body,
        grid=(E // B,),
        in_specs=[
            pl.BlockSpec((B, ED), lambda i: (i, 0)),
            pl.BlockSpec((ED, D), lambda i: (0, 0)),
            pl.BlockSpec((1, D), lambda i: (0, 0)),
        ],
        out_specs=pl.BlockSpec((B, D), lambda i: (i, 0)),
        out_shape=jax.ShapeDtypeStruct((EP, D), jnp.float32),
    )(edge_attr, W, b.reshape(1, D))


def _sc_message_pass(h, src2, dst2, t):
    N, D = h.shape
    EP = t.shape[0]
    NW = NC * NS
    EW = EP // NW
    NCH = EW // C
    NSC = NCH // SK
    NP = -(-(N + 1) // (NS * 8)) * (NS * 8)
    RPS = NP // NS
    JD = D // LANES
    assert NCH % UNR == 0 and RPS % 8 == 0
    NO = NCH // UNR

    mesh = plsc.VectorSubcoreMesh(core_axis_name="c", subcore_axis_name="s")

    @functools.partial(
        pl.kernel,
        out_type=jax.ShapeDtypeStruct((NC, NP, D), jnp.float32),
        mesh=mesh,
        scratch_types=(
            [pltpu.VMEM_SHARED((NP, D), jnp.float32)]
            + [pltpu.VMEM((SK, C), jnp.int32) for _ in range(2)]
            + [pltpu.VMEM((SK, C), jnp.int32) for _ in range(2)]
            + [pltpu.VMEM((C, D), jnp.float32) for _ in range(2)]
            + [pltpu.VMEM((C, D), jnp.float32) for _ in range(4)]
            + [pltpu.SemaphoreType.DMA for _ in range(8)]
        ),
    )
    def k(h_hbm, src_hbm, dst_hbm, t_hbm, out_hbm, acc_sh, *rest):
        srcv = rest[0:2]
        dstv = rest[2:4]
        tv = rest[4:6]
        rowsv = rest[6:10]
        sem_g = rest[10:12]
        sem_s = rest[12:14]
        sem_t = rest[14:16]
        sem_i = rest[16:18]

        cid = lax.axis_index("c")
        sid = lax.axis_index("s")
        wid = cid * NS + sid
        ebase = wid * EW
        cbase = wid * NCH
        row0 = sid * RPS

        def issue_idx(si, sl):
            row = cbase + si * SK
            pltpu.async_copy(src_hbm.at[pl.ds(row, SK)], srcv[sl], sem_i[sl])
            pltpu.async_copy(dst_hbm.at[pl.ds(row, SK)], dstv[sl], sem_i[sl])

        def wait_idx(sl):
            pltpu.make_async_copy(src_hbm.at[pl.ds(0, SK)], srcv[sl], sem_i[sl]).wait()
            pltpu.make_async_copy(dst_hbm.at[pl.ds(0, SK)], dstv[sl], sem_i[sl]).wait()

        def issue_t(c, sl):
            pltpu.async_copy(t_hbm.at[pl.ds(ebase + c * C, C)], tv[sl], sem_t[sl])

        def wait_t(sl):
            pltpu.make_async_copy(t_hbm.at[pl.ds(0, C)], tv[sl], sem_t[sl]).wait()

        def issue_gather(isl, krow, rsl, p):
            pltpu.async_copy(h_hbm.at[srcv[isl].at[krow]], rowsv[rsl], sem_g[p])

        def wait_gather(isl, krow, rsl, p):
            pltpu.make_async_copy(
                h_hbm.at[srcv[isl].at[krow]], rowsv[rsl], sem_g[p]).wait()

        def issue_scatter(isl, krow, rsl, p):
            pltpu.async_copy(rowsv[rsl], acc_sh.at[dstv[isl].at[krow]],
                             sem_s[p], add=True)

        def drain_scatter(isl, krow, rsl, p):
            pltpu.make_async_copy(rowsv[rsl], acc_sh.at[dstv[isl].at[krow]],
                                  sem_s[p]).wait()

        def compute(rsl, tsl):
            @pl.loop(0, C * JD, unroll=8)
            def _(g):
                i = g // JD
                jo = (g % JD) * LANES
                rowsv[rsl][i, pl.ds(jo, LANES)] = jnp.maximum(
                    rowsv[rsl][i, pl.ds(jo, LANES)]
                    + tv[tsl][i, pl.ds(jo, LANES)],
                    0.0,
                )

        zero = jnp.zeros((LANES,), jnp.float32)

        @pl.loop(0, C)
        def _(i):
            for j in range(JD):
                rowsv[0][i, pl.ds(j * LANES, LANES)] = zero

        @pl.loop(0, RPS // C)
        def _(r):
            pltpu.sync_copy(rowsv[0], acc_sh.at[pl.ds(row0 + r * C, C)])

        rem = RPS % C
        if rem:
            pltpu.sync_copy(
                rowsv[0].at[pl.ds(0, rem)],
                acc_sh.at[pl.ds(row0 + (RPS // C) * C, rem)],
            )
        plsc.subcore_barrier()

        issue_idx(0, 0)
        wait_idx(0)
        issue_t(0, 0)
        issue_t(1, 1)
        issue_gather(0, 0, 0, 0)
        issue_gather(0, 1, 1, 1)

        @pl.loop(0, NO)
        def _(o):
            for b in range(UNR):
                c = o * UNR + b
                rb = b % 4
                tb = b % 2
                isl = (b // SK) % 2
                krow = b % SK
                b2 = (b + 2) % UNR
                bm2 = (b - 2) % UNR

                wait_t(tb)
                wait_gather(isl, krow, rb, tb)
                compute(rb, tb)

                if b >= 2:
                    drain_scatter((bm2 // SK) % 2, bm2 % SK, bm2 % 4, tb)
                else:
                    @pl.when(o > 0)
                    def _():
                        drain_scatter((bm2 // SK) % 2, bm2 % SK, bm2 % 4, tb)

                if b < UNR - 2:
                    issue_t(c + 2, tb)
                else:
                    @pl.when(o < NO - 1)
                    def _():
                        issue_t(c + 2, tb)

                if krow == 1:
                    if b // SK == 0:
                        issue_idx(o * 2 + 1, 1)
                    else:
                        @pl.when(o < NO - 1)
                        def _():
                            issue_idx(o * 2 + 2, 0)

                if b < UNR - 2:
                    if b2 % SK == 0:
                        wait_idx((b2 // SK) % 2)
                    issue_gather((b2 // SK) % 2, b2 % SK, b2 % 4, tb)
                else:
                    @pl.when(o < NO - 1)
                    def _():
                        if b2 % SK == 0:
                            wait_idx((b2 // SK) % 2)
                        issue_gather((b2 // SK) % 2, b2 % SK, b2 % 4, tb)

                issue_scatter(isl, krow, rb, tb)

        for cc in (NCH - 2, NCH - 1):
            bb = cc % UNR
            drain_scatter((bb // SK) % 2, bb % SK, bb % 4, bb % 2)

        plsc.subcore_barrier()
        pltpu.sync_copy(
            acc_sh.at[pl.ds(row0, RPS)], out_hbm.at[cid, pl.ds(row0, RPS)]
        )

    return k(h, src2, dst2, t)


def _node_update(h, acc, W_nn, b_nn):
    N, D = h.shape
    B = 1000

    def body(h_ref, a_ref, w_ref, b_ref, o_ref):
        s = h_ref[...] + a_ref[0] + a_ref[1]
        z = jnp.dot(s, w_ref[...], preferred_element_type=jnp.float32) + b_ref[...]
        o_ref[...] = jnp.where(z >= 0, z, 0.01 * z)

    return pl.pallas_call(
        body,
        grid=(N // B,),
        in_specs=[
            pl.BlockSpec((B, D), lambda i: (i, 0)),
            pl.BlockSpec((NC, B, D), lambda i: (0, i, 0)),
            pl.BlockSpec((D, D), lambda i: (0, 0)),
            pl.BlockSpec((1, D), lambda i: (0, 0)),
        ],
        out_specs=pl.BlockSpec((B, D), lambda i: (i, 0)),
        out_shape=jax.ShapeDtypeStruct((N, D), jnp.float32),
    )(h, acc, W_nn, b_nn.reshape(1, D))


def _final(h, acc, W_nn, b_nn, W_dense, b_dense, W_out, b_out):
    N, D = h.shape
    DD = W_dense.shape[1]
    DO = W_out.shape[1]
    B = 1000

    def body(h_ref, a_ref, wn_ref, bn_ref, wd_ref, bd_ref, wo_ref, bo_ref,
             o_ref):
        s = h_ref[...] + a_ref[0] + a_ref[1]
        z = jnp.dot(s, wn_ref[...], preferred_element_type=jnp.float32) + bn_ref[...]
        z = jnp.where(z >= 0, z, 0.01 * z)
        z = jnp.dot(z, wd_ref[...], preferred_element_type=jnp.float32) + bd_ref[...]
        o_ref[...] = (
            jnp.dot(z, wo_ref[...], preferred_element_type=jnp.float32) + bo_ref[...]
        )

    return pl.pallas_call(
        body,
        grid=(N // B,),
        in_specs=[
            pl.BlockSpec((B, D), lambda i: (i, 0)),
            pl.BlockSpec((NC, B, D), lambda i: (0, i, 0)),
            pl.BlockSpec((D, D), lambda i: (0, 0)),
            pl.BlockSpec((1, D), lambda i: (0, 0)),
            pl.BlockSpec((D, DD), lambda i: (0, 0)),
            pl.BlockSpec((1, DD), lambda i: (0, 0)),
            pl.BlockSpec((DD, DO), lambda i: (0, 0)),
            pl.BlockSpec((1, DO), lambda i: (0, 0)),
        ],
        out_specs=pl.BlockSpec((B, DO), lambda i: (i, 0)),
        out_shape=jax.ShapeDtypeStruct((N, DO), jnp.float32),
    )(h, acc, W_nn, b_nn.reshape(1, D), W_dense, b_dense.reshape(1, DD),
      W_out, b_out.reshape(1, DO))


def kernel(x, edge_index, edge_attr, W_nn, b_nn, W_e0, b_e0, W_e1, b_e1,
           W_dense, b_dense, W_out, b_out):
    N = x.shape[0]
    E = edge_index.shape[1]
    EP = -(-E // (NC * NS * C * UNR)) * (NC * NS * C * UNR)
    pad = EP - E
    src2 = jnp.concatenate(
        [edge_index[0], jnp.zeros((pad,), jnp.int32)]).reshape(EP // C, C)
    dst2 = jnp.concatenate(
        [edge_index[1], jnp.full((pad,), N, jnp.int32)]).reshape(EP // C, C)
    t0 = _edge_linear(edge_attr, W_e0, b_e0, EP)
    t1 = _edge_linear(edge_attr, W_e1, b_e1, EP)
    acc0 = _sc_message_pass(x, src2, dst2, t0)
    h1 = _node_update(x, acc0, W_nn, b_nn)
    acc1 = _sc_message_pass(h1, src2, dst2, t1)
    return _final(h1, acc1, W_nn, b_nn, W_dense, b_dense, W_out, b_out)


# speedup vs baseline: 1.0179x; 1.0179x over previous
import functools

import jax
import jax.numpy as jnp
from jax import lax
from jax.experimental import pallas as pl
from jax.experimental.pallas import tpu as pltpu
from jax.experimental.pallas import tpu_sc as plsc

NC = 2
NS = 16
LANES = 16
C = 56
SK = 8
UNR = 2 * SK


def _edge_linear(edge_attr, W, b, EP):
    E, ED = edge_attr.shape
    D = W.shape[1]
    B = 2000

    def body(a_ref, w_ref, b_ref, o_ref):
        o_ref[...] = (
            jnp.dot(a_ref[...], w_ref[...], preferred_element_type=jnp.float32)
            + b_ref[...]
        )

    return pl.pallas_call(
        body,
        grid=(E // B,),
        in_specs=[
            pl.BlockSpec((B, ED), lambda i: (i, 0)),
            pl.BlockSpec((ED, D), lambda i: (0, 0)),
            pl.BlockSpec((1, D), lambda i: (0, 0)),
        ],
        out_specs=pl.BlockSpec((B, D), lambda i: (i, 0)),
        out_shape=jax.ShapeDtypeStruct((EP, D), jnp.float32),
    )(edge_attr, W, b.reshape(1, D))


def _sc_message_pass(h, src2, dst2, t):
    N, D = h.shape
    EP = t.shape[0]
    NW = NC * NS
    EW = EP // NW
    NCH = EW // C
    NSC = NCH // SK
    NP = -(-(N + 1) // (NS * 8)) * (NS * 8)
    RPS = NP // NS
    JD = D // LANES
    assert NCH % UNR == 0 and RPS % 8 == 0
    NO = NCH // UNR

    mesh = plsc.VectorSubcoreMesh(core_axis_name="c", subcore_axis_name="s")

    @functools.partial(
        pl.kernel,
        out_type=jax.ShapeDtypeStruct((NC, NP, D), jnp.float32),
        mesh=mesh,
        scratch_types=(
            [pltpu.VMEM_SHARED((NP, D), jnp.float32)]
            + [pltpu.VMEM((SK, C), jnp.int32) for _ in range(2)]
            + [pltpu.VMEM((SK, C), jnp.int32) for _ in range(2)]
            + [pltpu.VMEM((C, D), jnp.float32) for _ in range(2)]
            + [pltpu.VMEM((C, D), jnp.float32) for _ in range(4)]
            + [pltpu.SemaphoreType.DMA for _ in range(8)]
        ),
    )
    def k(h_hbm, src_hbm, dst_hbm, t_hbm, out_hbm, acc_sh, *rest):
        srcv = rest[0:2]
        dstv = rest[2:4]
        tv = rest[4:6]
        rowsv = rest[6:10]
        sem_g = rest[10:12]
        sem_s = rest[12:14]
        sem_t = rest[14:16]
        sem_i = rest[16:18]

        cid = lax.axis_index("c")
        sid = lax.axis_index("s")
        wid = cid * NS + sid
        ebase = wid * EW
        cbase = wid * NCH
        row0 = sid * RPS

        def issue_idx(si, sl):
            row = cbase + si * SK
            pltpu.async_copy(src_hbm.at[pl.ds(row, SK)], srcv[sl], sem_i[sl])
            pltpu.async_copy(dst_hbm.at[pl.ds(row, SK)], dstv[sl], sem_i[sl])

        def wait_idx(sl):
            pltpu.make_async_copy(src_hbm.at[pl.ds(0, SK)], srcv[sl], sem_i[sl]).wait()
            pltpu.make_async_copy(dst_hbm.at[pl.ds(0, SK)], dstv[sl], sem_i[sl]).wait()

        def issue_t(c, sl):
            pltpu.async_copy(t_hbm.at[pl.ds(ebase + c * C, C)], tv[sl], sem_t[sl])

        def wait_t(sl):
            pltpu.make_async_copy(t_hbm.at[pl.ds(0, C)], tv[sl], sem_t[sl]).wait()

        def issue_gather(isl, krow, rsl, p):
            pltpu.async_copy(h_hbm.at[srcv[isl].at[krow]], rowsv[rsl], sem_g[p])

        def wait_gather(isl, krow, rsl, p):
            pltpu.make_async_copy(
                h_hbm.at[srcv[isl].at[krow]], rowsv[rsl], sem_g[p]).wait()

        def issue_scatter(isl, krow, rsl, p):
            pltpu.async_copy(rowsv[rsl], acc_sh.at[dstv[isl].at[krow]],
                             sem_s[p], add=True)

        def drain_scatter(isl, krow, rsl, p):
            pltpu.make_async_copy(rowsv[rsl], acc_sh.at[dstv[isl].at[krow]],
                                  sem_s[p]).wait()

        def compute(rsl, tsl):
            @pl.loop(0, C * JD, unroll=8)
            def _(g):
                i = g // JD
                jo = (g % JD) * LANES
                rowsv[rsl][i, pl.ds(jo, LANES)] = jnp.maximum(
                    rowsv[rsl][i, pl.ds(jo, LANES)]
                    + tv[tsl][i, pl.ds(jo, LANES)],
                    0.0,
                )

        zero = jnp.zeros((LANES,), jnp.float32)

        @pl.loop(0, C)
        def _(i):
            for j in range(JD):
                rowsv[0][i, pl.ds(j * LANES, LANES)] = zero

        @pl.loop(0, RPS // C)
        def _(r):
            pltpu.sync_copy(rowsv[0], acc_sh.at[pl.ds(row0 + r * C, C)])

        rem = RPS % C
        if rem:
            pltpu.sync_copy(
                rowsv[0].at[pl.ds(0, rem)],
                acc_sh.at[pl.ds(row0 + (RPS // C) * C, rem)],
            )
        plsc.subcore_barrier()

        issue_idx(0, 0)
        wait_idx(0)
        issue_t(0, 0)
        issue_t(1, 1)
        issue_gather(0, 0, 0, 0)

        @pl.loop(0, NO)
        def _(o):
            for b in range(UNR):
                c = o * UNR + b
                rb = b % 4
                tb = b % 2
                isl = (b // SK) % 2
                krow = b % SK
                b1 = (b + 1) % UNR
                bm2 = (b - 2) % UNR

                wait_t(tb)
                wait_gather(isl, krow, rb, tb)

                if b < UNR - 1:
                    if b1 % SK == 0:
                        wait_idx((b1 // SK) % 2)
                    issue_gather((b1 // SK) % 2, b1 % SK, b1 % 4, 1 - tb)
                else:
                    @pl.when(o < NO - 1)
                    def _():
                        wait_idx(0)
                        issue_gather(0, 0, 0, 1 - tb)

                compute(rb, tb)

                if b >= 2:
                    drain_scatter((bm2 // SK) % 2, bm2 % SK, bm2 % 4, tb)
                else:
                    @pl.when(o > 0)
                    def _():
                        drain_scatter((bm2 // SK) % 2, bm2 % SK, bm2 % 4, tb)

                if b < UNR - 2:
                    issue_t(c + 2, tb)
                else:
                    @pl.when(o < NO - 1)
                    def _():
                        issue_t(c + 2, tb)

                if krow == 1:
                    if b // SK == 0:
                        issue_idx(o * 2 + 1, 1)
                    else:
                        @pl.when(o < NO - 1)
                        def _():
                            issue_idx(o * 2 + 2, 0)

                issue_scatter(isl, krow, rb, tb)

        for cc in (NCH - 2, NCH - 1):
            bb = cc % UNR
            drain_scatter((bb // SK) % 2, bb % SK, bb % 4, bb % 2)

        plsc.subcore_barrier()
        pltpu.sync_copy(
            acc_sh.at[pl.ds(row0, RPS)], out_hbm.at[cid, pl.ds(row0, RPS)]
        )

    return k(h, src2, dst2, t)


def _node_update(h, acc, W_nn, b_nn):
    N, D = h.shape
    B = 1000

    def body(h_ref, a_ref, w_ref, b_ref, o_ref):
        s = h_ref[...] + a_ref[0] + a_ref[1]
        z = jnp.dot(s, w_ref[...], preferred_element_type=jnp.float32) + b_ref[...]
        o_ref[...] = jnp.where(z >= 0, z, 0.01 * z)

    return pl.pallas_call(
        body,
        grid=(N // B,),
        in_specs=[
            pl.BlockSpec((B, D), lambda i: (i, 0)),
            pl.BlockSpec((NC, B, D), lambda i: (0, i, 0)),
            pl.BlockSpec((D, D), lambda i: (0, 0)),
            pl.BlockSpec((1, D), lambda i: (0, 0)),
        ],
        out_specs=pl.BlockSpec((B, D), lambda i: (i, 0)),
        out_shape=jax.ShapeDtypeStruct((N, D), jnp.float32),
    )(h, acc, W_nn, b_nn.reshape(1, D))


def _final(h, acc, W_nn, b_nn, W_dense, b_dense, W_out, b_out):
    N, D = h.shape
    DD = W_dense.shape[1]
    DO = W_out.shape[1]
    B = 1000

    def body(h_ref, a_ref, wn_ref, bn_ref, wd_ref, bd_ref, wo_ref, bo_ref,
             o_ref):
        s = h_ref[...] + a_ref[0] + a_ref[1]
        z = jnp.dot(s, wn_ref[...], preferred_element_type=jnp.float32) + bn_ref[...]
        z = jnp.where(z >= 0, z, 0.01 * z)
        z = jnp.dot(z, wd_ref[...], preferred_element_type=jnp.float32) + bd_ref[...]
        o_ref[...] = (
            jnp.dot(z, wo_ref[...], preferred_element_type=jnp.float32) + bo_ref[...]
        )

    return pl.pallas_call(
        body,
        grid=(N // B,),
        in_specs=[
            pl.BlockSpec((B, D), lambda i: (i, 0)),
            pl.BlockSpec((NC, B, D), lambda i: (0, i, 0)),
            pl.BlockSpec((D, D), lambda i: (0, 0)),
            pl.BlockSpec((1, D), lambda i: (0, 0)),
            pl.BlockSpec((D, DD), lambda i: (0, 0)),
            pl.BlockSpec((1, DD), lambda i: (0, 0)),
            pl.BlockSpec((DD, DO), lambda i: (0, 0)),
            pl.BlockSpec((1, DO), lambda i: (0, 0)),
        ],
        out_specs=pl.BlockSpec((B, DO), lambda i: (i, 0)),
        out_shape=jax.ShapeDtypeStruct((N, DO), jnp.float32),
    )(h, acc, W_nn, b_nn.reshape(1, D), W_dense, b_dense.reshape(1, DD),
      W_out, b_out.reshape(1, DO))


def kernel(x, edge_index, edge_attr, W_nn, b_nn, W_e0, b_e0, W_e1, b_e1,
           W_dense, b_dense, W_out, b_out):
    N = x.shape[0]
    E = edge_index.shape[1]
    EP = -(-E // (NC * NS * C * UNR)) * (NC * NS * C * UNR)
    pad = EP - E
    src2 = jnp.concatenate(
        [edge_index[0], jnp.zeros((pad,), jnp.int32)]).reshape(EP // C, C)
    dst2 = jnp.concatenate(
        [edge_index[1], jnp.full((pad,), N, jnp.int32)]).reshape(EP // C, C)
    t0 = _edge_linear(edge_attr, W_e0, b_e0, EP)
    t1 = _edge_linear(edge_attr, W_e1, b_e1, EP)
    acc0 = _sc_message_pass(x, src2, dst2, t0)
    h1 = _node_update(x, acc0, W_nn, b_nn)
    acc1 = _sc_message_pass(h1, src2, dst2, t1)
    return _final(h1, acc1, W_nn, b_nn, W_dense, b_dense, W_out, b_out)


# speedup vs baseline: 1.9983x; 1.9633x over previous
import functools

import jax
import jax.numpy as jnp
from jax import lax
from jax.experimental import pallas as pl
from jax.experimental.pallas import tpu as pltpu
from jax.experimental.pallas import tpu_sc as plsc

NC = 2
NS = 16
LANES = 16
C = 40
SK = 8
UNR = 2 * SK


def _edge_linear(edge_attr, W, b, EP):
    E, ED = edge_attr.shape
    D = W.shape[1]
    B = 2000

    def body(a_ref, w_ref, b_ref, o_ref):
        o_ref[...] = (
            jnp.dot(a_ref[...], w_ref[...], preferred_element_type=jnp.float32)
            + b_ref[...]
        )

    return pl.pallas_call(
        body,
        grid=(E // B,),
        in_specs=[
            pl.BlockSpec((B, ED), lambda i: (i, 0)),
            pl.BlockSpec((ED, D), lambda i: (0, 0)),
            pl.BlockSpec((1, D), lambda i: (0, 0)),
        ],
        out_specs=pl.BlockSpec((B, D), lambda i: (i, 0)),
        out_shape=jax.ShapeDtypeStruct((EP, D), jnp.float32),
    )(edge_attr, W, b.reshape(1, D))


def _sc_message_pass(h, src2, dst2, t):
    N, D = h.shape
    EP = t.shape[0]
    NW = NC * NS
    EW = EP // NW
    NCH = EW // C
    NSC = NCH // SK
    NP = -(-(N + 1) // (NS * 8)) * (NS * 8)
    RPS = NP // NS
    JD = D // LANES
    assert NCH % UNR == 0 and RPS % 8 == 0
    NO = NCH // UNR

    mesh = plsc.VectorSubcoreMesh(core_axis_name="c", subcore_axis_name="s")

    @functools.partial(
        pl.kernel,
        out_type=jax.ShapeDtypeStruct((NC, NP, D), jnp.float32),
        mesh=mesh,
        scratch_types=(
            [pltpu.VMEM_SHARED((NP, D), jnp.float32)]
            + [pltpu.VMEM((SK, C), jnp.int32) for _ in range(2)]
            + [pltpu.VMEM((SK, C), jnp.int32) for _ in range(2)]
            + [pltpu.VMEM((C, D), jnp.float32) for _ in range(2)]
            + [pltpu.VMEM((C, D), jnp.float32) for _ in range(4)]
            + [pltpu.SemaphoreType.DMA for _ in range(8)]
        ),
    )
    def k(h_hbm, src_hbm, dst_hbm, t_hbm, out_hbm, acc_sh, *rest):
        srcv = rest[0:2]
        dstv = rest[2:4]
        tv = rest[4:6]
        rowsv = rest[6:10]
        sem_g = rest[10:12]
        sem_s = rest[12:14]
        sem_t = rest[14:16]
        sem_i = rest[16:18]

        cid = lax.axis_index("c")
        sid = lax.axis_index("s")
        wid = cid * NS + sid
        ebase = wid * EW
        cbase = wid * NCH
        row0 = sid * RPS

        def issue_idx(si, sl):
            row = cbase + si * SK
            pltpu.async_copy(src_hbm.at[pl.ds(row, SK)], srcv[sl], sem_i[sl])
            pltpu.async_copy(dst_hbm.at[pl.ds(row, SK)], dstv[sl], sem_i[sl])

        def wait_idx(sl):
            pltpu.make_async_copy(src_hbm.at[pl.ds(0, SK)], srcv[sl], sem_i[sl]).wait()
            pltpu.make_async_copy(dst_hbm.at[pl.ds(0, SK)], dstv[sl], sem_i[sl]).wait()

        def issue_t(c, sl):
            pltpu.async_copy(t_hbm.at[pl.ds(ebase + c * C, C)], tv[sl], sem_t[sl])

        def wait_t(sl):
            pltpu.make_async_copy(t_hbm.at[pl.ds(0, C)], tv[sl], sem_t[sl]).wait()

        def issue_gather(isl, krow, rsl, p):
            pltpu.async_copy(h_hbm.at[srcv[isl].at[krow]], rowsv[rsl], sem_g[p])

        def wait_gather(isl, krow, rsl, p):
            pltpu.make_async_copy(
                h_hbm.at[srcv[isl].at[krow]], rowsv[rsl], sem_g[p]).wait()

        def issue_scatter(isl, krow, rsl, p):
            pltpu.async_copy(rowsv[rsl], acc_sh.at[dstv[isl].at[krow]],
                             sem_s[p], add=True)

        def drain_scatter(isl, krow, rsl, p):
            pltpu.make_async_copy(rowsv[rsl], acc_sh.at[dstv[isl].at[krow]],
                                  sem_s[p]).wait()

        def compute(rsl, tsl):
            @pl.loop(0, C * JD, unroll=8)
            def _(g):
                i = g // JD
                jo = (g % JD) * LANES
                rowsv[rsl][i, pl.ds(jo, LANES)] = jnp.maximum(
                    rowsv[rsl][i, pl.ds(jo, LANES)]
                    + tv[tsl][i, pl.ds(jo, LANES)],
                    0.0,
                )

        zero = jnp.zeros((LANES,), jnp.float32)

        @pl.loop(0, C)
        def _(i):
            for j in range(JD):
                rowsv[0][i, pl.ds(j * LANES, LANES)] = zero

        @pl.loop(0, RPS // C)
        def _(r):
            pltpu.sync_copy(rowsv[0], acc_sh.at[pl.ds(row0 + r * C, C)])

        rem = RPS % C
        if rem:
            pltpu.sync_copy(
                rowsv[0].at[pl.ds(0, rem)],
                acc_sh.at[pl.ds(row0 + (RPS // C) * C, rem)],
            )
        plsc.subcore_barrier()

        issue_idx(0, 0)
        wait_idx(0)
        issue_t(0, 0)
        issue_t(1, 1)
        issue_gather(0, 0, 0, 0)

        @pl.loop(0, NO)
        def _(o):
            for b in range(UNR):
                c = o * UNR + b
                rb = b % 4
                tb = b % 2
                isl = (b // SK) % 2
                krow = b % SK
                b1 = (b + 1) % UNR
                bm2 = (b - 2) % UNR

                wait_t(tb)
                wait_gather(isl, krow, rb, tb)

                if b < UNR - 1:
                    if b1 % SK == 0:
                        wait_idx((b1 // SK) % 2)
                    issue_gather((b1 // SK) % 2, b1 % SK, b1 % 4, 1 - tb)
                else:
                    @pl.when(o < NO - 1)
                    def _():
                        wait_idx(0)
                        issue_gather(0, 0, 0, 1 - tb)

                compute(rb, tb)

                if b >= 2:
                    drain_scatter((bm2 // SK) % 2, bm2 % SK, bm2 % 4, tb)
                else:
                    @pl.when(o > 0)
                    def _():
                        drain_scatter((bm2 // SK) % 2, bm2 % SK, bm2 % 4, tb)

                if b < UNR - 2:
                    issue_t(c + 2, tb)
                else:
                    @pl.when(o < NO - 1)
                    def _():
                        issue_t(c + 2, tb)

                if krow == 1:
                    if b // SK == 0:
                        issue_idx(o * 2 + 1, 1)
                    else:
                        @pl.when(o < NO - 1)
                        def _():
                            issue_idx(o * 2 + 2, 0)

                issue_scatter(isl, krow, rb, tb)

        for cc in (NCH - 2, NCH - 1):
            bb = cc % UNR
            drain_scatter((bb // SK) % 2, bb % SK, bb % 4, bb % 2)

        plsc.subcore_barrier()
        pltpu.sync_copy(
            acc_sh.at[pl.ds(row0, RPS)], out_hbm.at[cid, pl.ds(row0, RPS)]
        )

    return k(h, src2, dst2, t)


def _node_update(h, acc, W_nn, b_nn):
    N, D = h.shape
    B = 1000

    def body(h_ref, a_ref, w_ref, b_ref, o_ref):
        s = h_ref[...] + a_ref[0] + a_ref[1]
        z = jnp.dot(s, w_ref[...], preferred_element_type=jnp.float32) + b_ref[...]
        o_ref[...] = jnp.where(z >= 0, z, 0.01 * z)

    return pl.pallas_call(
        body,
        grid=(N // B,),
        in_specs=[
            pl.BlockSpec((B, D), lambda i: (i, 0)),
            pl.BlockSpec((NC, B, D), lambda i: (0, i, 0)),
            pl.BlockSpec((D, D), lambda i: (0, 0)),
            pl.BlockSpec((1, D), lambda i: (0, 0)),
        ],
        out_specs=pl.BlockSpec((B, D), lambda i: (i, 0)),
        out_shape=jax.ShapeDtypeStruct((N, D), jnp.float32),
    )(h, acc, W_nn, b_nn.reshape(1, D))


def _final(h, acc, W_nn, b_nn, W_dense, b_dense, W_out, b_out):
    N, D = h.shape
    DD = W_dense.shape[1]
    DO = W_out.shape[1]
    B = 1000

    def body(h_ref, a_ref, wn_ref, bn_ref, wd_ref, bd_ref, wo_ref, bo_ref,
             o_ref):
        s = h_ref[...] + a_ref[0] + a_ref[1]
        z = jnp.dot(s, wn_ref[...], preferred_element_type=jnp.float32) + bn_ref[...]
        z = jnp.where(z >= 0, z, 0.01 * z)
        z = jnp.dot(z, wd_ref[...], preferred_element_type=jnp.float32) + bd_ref[...]
        o_ref[...] = (
            jnp.dot(z, wo_ref[...], preferred_element_type=jnp.float32) + bo_ref[...]
        )

    return pl.pallas_call(
        body,
        grid=(N // B,),
        in_specs=[
            pl.BlockSpec((B, D), lambda i: (i, 0)),
            pl.BlockSpec((NC, B, D), lambda i: (0, i, 0)),
            pl.BlockSpec((D, D), lambda i: (0, 0)),
            pl.BlockSpec((1, D), lambda i: (0, 0)),
            pl.BlockSpec((D, DD), lambda i: (0, 0)),
            pl.BlockSpec((1, DD), lambda i: (0, 0)),
            pl.BlockSpec((DD, DO), lambda i: (0, 0)),
            pl.BlockSpec((1, DO), lambda i: (0, 0)),
        ],
        out_specs=pl.BlockSpec((B, DO), lambda i: (i, 0)),
        out_shape=jax.ShapeDtypeStruct((N, DO), jnp.float32),
    )(h, acc, W_nn, b_nn.reshape(1, D), W_dense, b_dense.reshape(1, DD),
      W_out, b_out.reshape(1, DO))


def kernel(x, edge_index, edge_attr, W_nn, b_nn, W_e0, b_e0, W_e1, b_e1,
           W_dense, b_dense, W_out, b_out):
    N = x.shape[0]
    E = edge_index.shape[1]
    EP = -(-E // (NC * NS * C * UNR)) * (NC * NS * C * UNR)
    pad = EP - E
    src2 = jnp.concatenate(
        [edge_index[0], jnp.zeros((pad,), jnp.int32)]).reshape(EP // C, C)
    dst2 = jnp.concatenate(
        [edge_index[1], jnp.full((pad,), N, jnp.int32)]).reshape(EP // C, C)
    t0 = _edge_linear(edge_attr, W_e0, b_e0, EP)
    t1 = _edge_linear(edge_attr, W_e1, b_e1, EP)
    acc0 = _sc_message_pass(x, src2, dst2, t0)
    h1 = _node_update(x, acc0, W_nn, b_nn)
    acc1 = _sc_message_pass(h1, src2, dst2, t1)
    return _final(h1, acc1, W_nn, b_nn, W_dense, b_dense, W_out, b_out)


# speedup vs baseline: 2.3723x; 1.1871x over previous
import functools

import jax
import jax.numpy as jnp
from jax import lax
from jax.experimental import pallas as pl
from jax.experimental.pallas import tpu as pltpu
from jax.experimental.pallas import tpu_sc as plsc

NC = 2
NS = 16
LANES = 16
C = 40
NB = 4


def _edge_linear(edge_attr, W, b, EP):
    E, ED = edge_attr.shape
    D = W.shape[1]
    B = 2000

    def body(a_ref, w_ref, b_ref, o_ref):
        o_ref[...] = (
            jnp.dot(a_ref[...], w_ref[...], preferred_element_type=jnp.float32)
            + b_ref[...]
        )

    return pl.pallas_call(
        body,
        grid=(E // B,),
        in_specs=[
            pl.BlockSpec((B, ED), lambda i: (i, 0)),
            pl.BlockSpec((ED, D), lambda i: (0, 0)),
            pl.BlockSpec((1, D), lambda i: (0, 0)),
        ],
        out_specs=pl.BlockSpec((B, D), lambda i: (i, 0)),
        out_shape=jax.ShapeDtypeStruct((EP, D), jnp.float32),
    )(edge_attr, W, b.reshape(1, D))


def _sc_message_pass(h, src, dst, t):
    N, D = h.shape
    EP = src.shape[0]
    NW = NC * NS
    EW = EP // NW
    NCH = EW // C
    NP = -(-(N + 1) // (NS * 8)) * (NS * 8)
    RPS = NP // NS
    JD = D // LANES
    assert NCH % NB == 0 and RPS % 8 == 0

    mesh = plsc.VectorSubcoreMesh(core_axis_name="c", subcore_axis_name="s")

    @functools.partial(
        pl.kernel,
        out_type=jax.ShapeDtypeStruct((NC, NP, D), jnp.float32),
        mesh=mesh,
        scratch_types=(
            [pltpu.VMEM_SHARED((NP, D), jnp.float32)]
            + [pltpu.VMEM((C,), jnp.int32) for _ in range(NB)]
            + [pltpu.VMEM((C,), jnp.int32) for _ in range(NB)]
            + [pltpu.VMEM((C, D), jnp.float32) for _ in range(NB)]
            + [pltpu.VMEM((C, D), jnp.float32) for _ in range(NB)]
            + [pltpu.SemaphoreType.DMA, pltpu.SemaphoreType.DMA]
            + [pltpu.SemaphoreType.DMA for _ in range(NB)]
        ),
    )
    def k(h_hbm, src_hbm, dst_hbm, t_hbm, out_hbm, acc_sh, *rest):
        srcv = rest[0:NB]
        dstv = rest[NB:2 * NB]
        tv = rest[2 * NB:3 * NB]
        rowsv = rest[3 * NB:4 * NB]
        sem_ld = rest[4 * NB]
        sem_g = rest[4 * NB + 1]
        sem_s = rest[4 * NB + 2:4 * NB + 2 + NB]

        cid = lax.axis_index("c")
        sid = lax.axis_index("s")
        wid = cid * NS + sid
        base = wid * EW
        row0 = sid * RPS

        def issue_loads(c, b):
            off = base + c * C
            pltpu.async_copy(src_hbm.at[pl.ds(off, C)], srcv[b], sem_ld)
            pltpu.async_copy(dst_hbm.at[pl.ds(off, C)], dstv[b], sem_ld)
            pltpu.async_copy(t_hbm.at[pl.ds(off, C)], tv[b], sem_ld)

        def wait_loads(b):
            pltpu.make_async_copy(src_hbm.at[pl.ds(0, C)], srcv[b], sem_ld).wait()
            pltpu.make_async_copy(dst_hbm.at[pl.ds(0, C)], dstv[b], sem_ld).wait()
            pltpu.make_async_copy(t_hbm.at[pl.ds(0, C)], tv[b], sem_ld).wait()

        def issue_gather(b):
            pltpu.async_copy(h_hbm.at[srcv[b]], rowsv[b], sem_g)

        def wait_gather(b):
            pltpu.make_async_copy(h_hbm.at[srcv[b]], rowsv[b], sem_g).wait()

        def issue_scatter(b):
            pltpu.async_copy(rowsv[b], acc_sh.at[dstv[b]], sem_s[b], add=True)

        def drain_scatter(b):
            pltpu.make_async_copy(rowsv[b], acc_sh.at[dstv[b]], sem_s[b]).wait()

        zero = jnp.zeros((LANES,), jnp.float32)

        @pl.loop(0, C)
        def _(i):
            for j in range(JD):
                rowsv[0][i, pl.ds(j * LANES, LANES)] = zero

        @pl.loop(0, RPS // C)
        def _(r):
            pltpu.sync_copy(rowsv[0], acc_sh.at[pl.ds(row0 + r * C, C)])

        rem = RPS % C
        if rem:
            pltpu.sync_copy(
                rowsv[0].at[pl.ds(0, rem)],
                acc_sh.at[pl.ds(row0 + (RPS // C) * C, rem)],
            )
        plsc.subcore_barrier()

        issue_loads(0, 0)
        wait_loads(0)
        issue_gather(0)
        issue_loads(1, 1)

        @pl.loop(0, NCH // NB)
        def _(o):
            for b in range(NB):
                c = o * NB + b
                s1 = (b + 1) % NB
                s2 = (b + 2) % NB

                @pl.when(c + 1 < NCH)
                def _():
                    wait_loads(s1)
                wait_gather(b)

                @pl.when(c + 1 < NCH)
                def _():
                    issue_gather(s1)

                @pl.when(c >= NB - 2)
                def _():
                    drain_scatter(s2)

                @pl.when(c + 2 < NCH)
                def _():
                    issue_loads(c + 2, s2)

                @pl.loop(0, C * JD, unroll=8)
                def _(g):
                    i = g // JD
                    jo = (g % JD) * LANES
                    rowsv[b][i, pl.ds(jo, LANES)] = jnp.maximum(
                        rowsv[b][i, pl.ds(jo, LANES)] + tv[b][i, pl.ds(jo, LANES)],
                        0.0,
                    )

                issue_scatter(b)

        for c in range(NCH - (NB - 2), NCH):
            drain_scatter(c % NB)

        plsc.subcore_barrier()
        pltpu.sync_copy(
            acc_sh.at[pl.ds(row0, RPS)], out_hbm.at[cid, pl.ds(row0, RPS)]
        )

    return k(h, src, dst, t)


def _node_update(h, acc, W_nn, b_nn):
    N, D = h.shape
    B = 1000

    def body(h_ref, a_ref, w_ref, b_ref, o_ref):
        s = h_ref[...] + a_ref[0] + a_ref[1]
        z = jnp.dot(s, w_ref[...], preferred_element_type=jnp.float32) + b_ref[...]
        o_ref[...] = jnp.where(z >= 0, z, 0.01 * z)

    return pl.pallas_call(
        body,
        grid=(N // B,),
        in_specs=[
            pl.BlockSpec((B, D), lambda i: (i, 0)),
            pl.BlockSpec((NC, B, D), lambda i: (0, i, 0)),
            pl.BlockSpec((D, D), lambda i: (0, 0)),
            pl.BlockSpec((1, D), lambda i: (0, 0)),
        ],
        out_specs=pl.BlockSpec((B, D), lambda i: (i, 0)),
        out_shape=jax.ShapeDtypeStruct((N, D), jnp.float32),
    )(h, acc, W_nn, b_nn.reshape(1, D))


def _final(h, acc, W_nn, b_nn, W_dense, b_dense, W_out, b_out):
    N, D = h.shape
    DD = W_dense.shape[1]
    DO = W_out.shape[1]
    B = 1000

    def body(h_ref, a_ref, wn_ref, bn_ref, wd_ref, bd_ref, wo_ref, bo_ref,
             o_ref):
        s = h_ref[...] + a_ref[0] + a_ref[1]
        z = jnp.dot(s, wn_ref[...], preferred_element_type=jnp.float32) + bn_ref[...]
        z = jnp.where(z >= 0, z, 0.01 * z)
        z = jnp.dot(z, wd_ref[...], preferred_element_type=jnp.float32) + bd_ref[...]
        o_ref[...] = (
            jnp.dot(z, wo_ref[...], preferred_element_type=jnp.float32) + bo_ref[...]
        )

    return pl.pallas_call(
        body,
        grid=(N // B,),
        in_specs=[
            pl.BlockSpec((B, D), lambda i: (i, 0)),
            pl.BlockSpec((NC, B, D), lambda i: (0, i, 0)),
            pl.BlockSpec((D, D), lambda i: (0, 0)),
            pl.BlockSpec((1, D), lambda i: (0, 0)),
            pl.BlockSpec((D, DD), lambda i: (0, 0)),
            pl.BlockSpec((1, DD), lambda i: (0, 0)),
            pl.BlockSpec((DD, DO), lambda i: (0, 0)),
            pl.BlockSpec((1, DO), lambda i: (0, 0)),
        ],
        out_specs=pl.BlockSpec((B, DO), lambda i: (i, 0)),
        out_shape=jax.ShapeDtypeStruct((N, DO), jnp.float32),
    )(h, acc, W_nn, b_nn.reshape(1, D), W_dense, b_dense.reshape(1, DD),
      W_out, b_out.reshape(1, DO))


def kernel(x, edge_index, edge_attr, W_nn, b_nn, W_e0, b_e0, W_e1, b_e1,
           W_dense, b_dense, W_out, b_out):
    N = x.shape[0]
    E = edge_index.shape[1]
    EP = -(-E // (NC * NS * C * NB)) * (NC * NS * C * NB)
    pad = EP - E
    src = jnp.concatenate([edge_index[0], jnp.zeros((pad,), jnp.int32)])
    dst = jnp.concatenate([edge_index[1], jnp.full((pad,), N, jnp.int32)])
    t0 = _edge_linear(edge_attr, W_e0, b_e0, EP)
    acc0 = _sc_message_pass(x, src, dst, t0)
    t1 = _edge_linear(edge_attr, W_e1, b_e1, EP)
    h1 = _node_update(x, acc0, W_nn, b_nn)
    acc1 = _sc_message_pass(h1, src, dst, t1)
    return _final(h1, acc1, W_nn, b_nn, W_dense, b_dense, W_out, b_out)
